# Initial kernel scaffold; baseline (speedup 1.0000x reference)
import functools

import jax
import jax.numpy as jnp
from jax import lax
from jax.experimental import pallas as pl
from jax.experimental.pallas import tpu as pltpu
from jax.experimental.pallas import tpu_sc as plsc

NC, NS = 2, 16
NW = NC * NS
B = 16384 * 50
D = 32
BPW = B // NW
CHUNK = 3200
NCHUNK = BPW // CHUNK

_mesh = plsc.VectorSubcoreMesh(core_axis_name="c", subcore_axis_name="s")


@functools.partial(
    pl.kernel,
    out_type=jax.ShapeDtypeStruct((B, D), jnp.float32),
    mesh=_mesh,
    scratch_types=[
        pltpu.VMEM((CHUNK,), jnp.int32),
        pltpu.VMEM((CHUNK, D), jnp.float32),
        pltpu.SemaphoreType.DMA,
    ],
)
def _gather_kernel(idx_hbm, table_hbm, out_hbm, idx_v, rows_v, sem):
    wid = lax.axis_index("s") * NC + lax.axis_index("c")
    base = wid * BPW

    @pl.loop(0, NCHUNK)
    def _chunk(ci):
        off = base + ci * CHUNK
        pltpu.sync_copy(idx_hbm.at[pl.ds(off, CHUNK)], idx_v)
        pltpu.async_copy(table_hbm.at[idx_v], rows_v, sem).wait()
        pltpu.sync_copy(rows_v, out_hbm.at[pl.ds(off, CHUNK)])


def kernel(input, item_embedding):
    flat = input.reshape(-1).astype(jnp.int32)
    out = _gather_kernel(flat, item_embedding)
    return out.reshape(input.shape + (item_embedding.shape[-1],))


# speedup vs baseline: 1.1110x; 1.1110x over previous
import functools

import jax
import jax.numpy as jnp
from jax import lax
from jax.experimental import pallas as pl
from jax.experimental.pallas import tpu as pltpu
from jax.experimental.pallas import tpu_sc as plsc

NC, NS = 2, 16
NW = NC * NS
B = 16384 * 50
D = 32
BPW = B // NW
CHUNK = 3200
NCHUNK = BPW // CHUNK

_mesh = plsc.VectorSubcoreMesh(core_axis_name="c", subcore_axis_name="s")


@functools.partial(
    pl.kernel,
    out_type=jax.ShapeDtypeStruct((B, D), jnp.float32),
    mesh=_mesh,
    scratch_types=[
        pltpu.VMEM((CHUNK,), jnp.int32),
        pltpu.VMEM((CHUNK, D), jnp.float32),
        pltpu.SemaphoreType.DMA,
    ],
    compiler_params=pltpu.CompilerParams(use_tc_tiling_on_sc=False),
)
def _gather_kernel(idx_hbm, table_hbm, out_hbm, idx_v, rows_v, sem):
    wid = lax.axis_index("s") * NC + lax.axis_index("c")
    base = wid * BPW

    @pl.loop(0, NCHUNK)
    def _chunk(ci):
        off = base + ci * CHUNK
        pltpu.sync_copy(idx_hbm.at[pl.ds(off, CHUNK)], idx_v)
        pltpu.async_copy(table_hbm.at[idx_v], rows_v, sem).wait()
        pltpu.sync_copy(rows_v, out_hbm.at[pl.ds(off, CHUNK)])


def kernel(input, item_embedding):
    flat = input.reshape(-1).astype(jnp.int32)
    out = _gather_kernel(flat, item_embedding)
    return out.reshape(input.shape + (item_embedding.shape[-1],))


# speedup vs baseline: 1.1140x; 1.0027x over previous
import functools

import jax
import jax.numpy as jnp
from jax import lax
from jax.experimental import pallas as pl
from jax.experimental.pallas import tpu as pltpu
from jax.experimental.pallas import tpu_sc as plsc

NC, NS = 2, 16
NW = NC * NS
B = 16384 * 50
D = 32
BPW = B // NW
C = 1600
NB = 2
NCH = BPW // C

_mesh = plsc.VectorSubcoreMesh(core_axis_name="c", subcore_axis_name="s")


@functools.partial(
    pl.kernel,
    out_type=jax.ShapeDtypeStruct((B, D), jnp.float32),
    mesh=_mesh,
    scratch_types=[
        pltpu.VMEM((BPW,), jnp.int32),
        pltpu.VMEM((NB, C, D), jnp.float32),
        pltpu.SemaphoreType.DMA,
        pltpu.SemaphoreType.DMA,
        pltpu.SemaphoreType.DMA,
        pltpu.SemaphoreType.DMA,
    ],
    compiler_params=pltpu.CompilerParams(use_tc_tiling_on_sc=False),
)
def _gather_kernel(idx_hbm, table_hbm, out_hbm, idx_v, rows_v, g0, g1, s0, s1):
    wid = lax.axis_index("s") * NC + lax.axis_index("c")
    base = wid * BPW
    gsem = (g0, g1)
    ssem = (s0, s1)

    pltpu.sync_copy(idx_hbm.at[pl.ds(base, BPW)], idx_v)

    def fire_gather(i, b):
        pltpu.async_copy(
            table_hbm.at[idx_v.at[pl.ds(i * C, C)]], rows_v.at[b], gsem[b])

    def wait_gather(i, b):
        pltpu.make_async_copy(
            table_hbm.at[idx_v.at[pl.ds(i * C, C)]], rows_v.at[b], gsem[b]).wait()

    def fire_store(i, b):
        pltpu.async_copy(
            rows_v.at[b], out_hbm.at[pl.ds(base + i * C, C)], ssem[b])

    def wait_store(i, b):
        pltpu.make_async_copy(
            rows_v.at[b], out_hbm.at[pl.ds(base + i * C, C)], ssem[b]).wait()

    for b in range(NB):
        fire_gather(b, b)

    @pl.loop(0, NCH, step=NB)
    def _chunk(ci):
        for b in range(NB):
            i = ci + b
            wait_gather(i, b)
            fire_store(i, b)

            @pl.when(i + NB < NCH)
            def _refill():
                wait_store(i, b)
                fire_gather(i + NB, b)

    for b in range(NB):
        wait_store(NCH - NB + b, b)


def kernel(input, item_embedding):
    flat = input.reshape(-1).astype(jnp.int32)
    out = _gather_kernel(flat, item_embedding)
    return out.reshape(input.shape + (item_embedding.shape[-1],))


# speedup vs baseline: 1.4839x; 1.3320x over previous
import functools

import jax
import jax.numpy as jnp
from jax import lax
from jax.experimental import pallas as pl
from jax.experimental.pallas import tpu as pltpu
from jax.experimental.pallas import tpu_sc as plsc

NC, NS = 2, 16
NW = NC * NS
NB_SAMPLES = 16384
S = 50
D = 32
SPW = NB_SAMPLES // NW
IPW = SPW * S

_mesh = plsc.VectorSubcoreMesh(core_axis_name="c", subcore_axis_name="s")


@functools.partial(
    pl.kernel,
    out_type=jax.ShapeDtypeStruct((S, D, NB_SAMPLES), jnp.float32),
    mesh=_mesh,
    scratch_types=[
        pltpu.VMEM((IPW,), jnp.int32),
        pltpu.VMEM((IPW,), jnp.int32),
        pltpu.VMEM((2, SPW, D), jnp.float32),
        pltpu.VMEM((2, D, SPW), jnp.float32),
        pltpu.SemaphoreType.DMA,
        pltpu.SemaphoreType.DMA,
        pltpu.SemaphoreType.DMA,
        pltpu.SemaphoreType.DMA,
    ],
    compiler_params=pltpu.CompilerParams(
        use_tc_tiling_on_sc=False, needs_layout_passes=False),
)
def _gather_kernel(idx_hbm, table_hbm, out_hbm, idx_v, idxt_v, g_v, t_v,
                   g0, g1, s0, s1):
    wid = lax.axis_index("s") * NC + lax.axis_index("c")
    b0 = wid * SPW
    gsem = (g0, g1)
    ssem = (s0, s1)

    pltpu.sync_copy(idx_hbm.at[pl.ds(b0 * S, IPW)], idx_v)

    lanes = lax.iota(jnp.int32, 16)
    riota_s = lanes * S
    riota_d = lanes * D

    @pl.loop(0, S)
    def _slot(s):
        @pl.loop(0, SPW // 16)
        def _grp(g):
            vec = riota_s + (g * (16 * S) + s)
            idxt_v[pl.ds(s * SPW + g * 16, 16)] = plsc.load_gather(idx_v, [vec])

    def fire_gather(s, b):
        pltpu.async_copy(
            table_hbm.at[idxt_v.at[pl.ds(s * SPW, SPW)]], g_v.at[b], gsem[b])

    def wait_gather(s, b):
        pltpu.make_async_copy(
            table_hbm.at[idxt_v.at[pl.ds(s * SPW, SPW)]], g_v.at[b], gsem[b]).wait()

    def drain_stores(b):
        for d in range(D):
            pltpu.make_async_copy(
                t_v.at[b, d], out_hbm.at[0, d, pl.ds(b0, SPW)], ssem[b]).wait()

    fire_gather(0, 0)

    @pl.loop(0, S, step=2)
    def _pipe(si):
        for b in range(2):
            s = si + b
            wait_gather(s, b)

            @pl.when(s + 1 < S)
            def _next():
                fire_gather(s + 1, 1 - b)

            @pl.when(s >= 2)
            def _drain():
                drain_stores(b)

            @pl.loop(0, SPW // 16)
            def _grp(g):
                rvec = lanes + g * 16
                for d in range(D):
                    cvec = jnp.full((16,), d, jnp.int32)
                    t_v[b, d, pl.ds(g * 16, 16)] = plsc.load_gather(
                        g_v.at[b], [rvec, cvec])

            for d in range(D):
                pltpu.async_copy(
                    t_v.at[b, d], out_hbm.at[s, d, pl.ds(b0, SPW)], ssem[b])

    for b in range(2):
        drain_stores(b)


def kernel(input, item_embedding):
    flat = input.reshape(-1).astype(jnp.int32)
    out = _gather_kernel(flat, item_embedding)
    return jnp.transpose(out, (2, 0, 1))


# speedup vs baseline: 1.4856x; 1.0011x over previous
import functools

import jax
import jax.numpy as jnp
from jax import lax
from jax.experimental import pallas as pl
from jax.experimental.pallas import tpu as pltpu
from jax.experimental.pallas import tpu_sc as plsc

NC, NS = 2, 16
NW = NC * NS
NB_SAMPLES = 16384
S = 50
D = 32
SPW = NB_SAMPLES // NW
IPW = SPW * S

_mesh = plsc.VectorSubcoreMesh(core_axis_name="c", subcore_axis_name="s")


@functools.partial(
    pl.kernel,
    out_type=jax.ShapeDtypeStruct((S, D, NB_SAMPLES), jnp.float32),
    mesh=_mesh,
    scratch_types=[
        pltpu.VMEM((SPW, S), jnp.int32),
        pltpu.VMEM((IPW,), jnp.int32),
        pltpu.VMEM((2, SPW, D), jnp.float32),
        pltpu.VMEM((2, D, SPW), jnp.float32),
        pltpu.SemaphoreType.DMA,
        pltpu.SemaphoreType.DMA,
        pltpu.SemaphoreType.DMA,
        pltpu.SemaphoreType.DMA,
    ],
    compiler_params=pltpu.CompilerParams(
        use_tc_tiling_on_sc=False, needs_layout_passes=False),
)
def _gather_kernel(idx_hbm, table_hbm, out_hbm, idx_v, idxt_v, g_v, t_v,
                   g0, g1, s0, s1):
    wid = lax.axis_index("s") * NC + lax.axis_index("c")
    b0 = wid * SPW
    gsem = (g0, g1)
    ssem = (s0, s1)

    pltpu.sync_copy(idx_hbm.at[pl.ds(b0, SPW), :], idx_v)

    lanes = lax.iota(jnp.int32, 16)
    riota_d = lanes * D

    @pl.loop(0, S)
    def _slot(s):
        svec = jnp.full((16,), 0, jnp.int32) + s

        @pl.loop(0, SPW // 16)
        def _grp(g):
            rvec = lanes + g * 16
            idxt_v[pl.ds(s * SPW + g * 16, 16)] = plsc.load_gather(
                idx_v, [rvec, svec])

    def fire_gather(s, b):
        pltpu.async_copy(
            table_hbm.at[idxt_v.at[pl.ds(s * SPW, SPW)]], g_v.at[b], gsem[b])

    def wait_gather(s, b):
        pltpu.make_async_copy(
            table_hbm.at[idxt_v.at[pl.ds(s * SPW, SPW)]], g_v.at[b], gsem[b]).wait()

    def drain_stores(b):
        pltpu.make_async_copy(
            t_v.at[b], out_hbm.at[0, :, pl.ds(b0, SPW)], ssem[b]).wait()

    fire_gather(0, 0)

    @pl.loop(0, S, step=2)
    def _pipe(si):
        for b in range(2):
            s = si + b
            wait_gather(s, b)

            @pl.when(s + 1 < S)
            def _next():
                fire_gather(s + 1, 1 - b)

            @pl.when(s >= 2)
            def _drain():
                drain_stores(b)

            @pl.loop(0, SPW // 16)
            def _grp(g):
                rvec = lanes + g * 16
                for d in range(D):
                    cvec = jnp.full((16,), d, jnp.int32)
                    t_v[b, d, pl.ds(g * 16, 16)] = plsc.load_gather(
                        g_v.at[b], [rvec, cvec])

            pltpu.async_copy(
                t_v.at[b], out_hbm.at[s, :, pl.ds(b0, SPW)], ssem[b])

    for b in range(2):
        drain_stores(b)


def kernel(input, item_embedding):
    out = _gather_kernel(input, item_embedding)
    return jnp.transpose(out, (2, 0, 1))


# speedup vs baseline: 1.8729x; 1.2607x over previous
import functools

import jax
import jax.numpy as jnp
from jax import lax
from jax.experimental import pallas as pl
from jax.experimental.pallas import tpu as pltpu
from jax.experimental.pallas import tpu_sc as plsc

NC, NS = 2, 16
NW = NC * NS
NB_SAMPLES = 16384
S = 50
D = 32
SPW = NB_SAMPLES // NW
IPW = SPW * S

_mesh = plsc.VectorSubcoreMesh(core_axis_name="c", subcore_axis_name="s")


@functools.partial(
    pl.kernel,
    out_type=jax.ShapeDtypeStruct((S, D, NB_SAMPLES), jnp.float32),
    mesh=_mesh,
    scratch_types=[
        pltpu.VMEM((SPW, S), jnp.int32),
        pltpu.VMEM((IPW,), jnp.int32),
        pltpu.VMEM((2, SPW, D), jnp.float32),
        pltpu.VMEM((2, D * SPW), jnp.float32),
        pltpu.SemaphoreType.DMA,
        pltpu.SemaphoreType.DMA,
        pltpu.SemaphoreType.DMA,
        pltpu.SemaphoreType.DMA,
    ],
    compiler_params=pltpu.CompilerParams(
        use_tc_tiling_on_sc=False, needs_layout_passes=False),
)
def _gather_kernel(idx_hbm, table_hbm, out_hbm, idx_v, idxt_v, g_v, t_v,
                   g0, g1, s0, s1):
    wid = lax.axis_index("s") * NC + lax.axis_index("c")
    b0 = wid * SPW
    gsem = (g0, g1)
    ssem = (s0, s1)

    pltpu.sync_copy(idx_hbm.at[pl.ds(b0, SPW), :], idx_v)

    lanes = lax.iota(jnp.int32, 16)
    scat_lo = lanes * SPW
    scat_hi = (lanes + 16) * SPW

    @pl.loop(0, S)
    def _slot(s):
        svec = jnp.full((16,), 0, jnp.int32) + s

        @plsc.parallel_loop(0, SPW // 16)
        def _grp(g):
            rvec = lanes + g * 16
            idxt_v[pl.ds(s * SPW + g * 16, 16)] = plsc.load_gather(
                idx_v, [rvec, svec])

    def fire_gather(s, b):
        pltpu.async_copy(
            table_hbm.at[idxt_v.at[pl.ds(s * SPW, SPW)]], g_v.at[b], gsem[b])

    def wait_gather(s, b):
        pltpu.make_async_copy(
            table_hbm.at[idxt_v.at[pl.ds(s * SPW, SPW)]], g_v.at[b], gsem[b]).wait()

    def drain_stores(b):
        for d in range(D):
            pltpu.make_async_copy(
                t_v.at[b, pl.ds(d * SPW, SPW)],
                out_hbm.at[0, d, pl.ds(b0, SPW)], ssem[b]).wait()

    fire_gather(0, 0)

    @pl.loop(0, S, step=2)
    def _pipe(si):
        for b in range(2):
            s = si + b
            wait_gather(s, b)

            @pl.when(s + 1 < S)
            def _next():
                fire_gather(s + 1, 1 - b)

            @pl.when(s >= 2)
            def _drain():
                drain_stores(b)

            @plsc.parallel_loop(0, SPW, unroll=8)
            def _row(r):
                plsc.store_scatter(t_v.at[b], [scat_lo + r],
                                   g_v[b, r, pl.ds(0, 16)])
                plsc.store_scatter(t_v.at[b], [scat_hi + r],
                                   g_v[b, r, pl.ds(16, 16)])

            for d in range(D):
                pltpu.async_copy(
                    t_v.at[b, pl.ds(d * SPW, SPW)],
                    out_hbm.at[s, d, pl.ds(b0, SPW)], ssem[b])

    for b in range(2):
        drain_stores(b)


def kernel(input, item_embedding):
    out = _gather_kernel(input, item_embedding)
    return jnp.transpose(out, (2, 0, 1))
